# baseline (speedup 1.0000x reference)
import functools

import jax
import jax.numpy as jnp
from jax import lax
from jax.experimental import pallas as pl
from jax.experimental.pallas import tpu as pltpu
from jax.experimental.pallas import tpu_sc as plsc

D = 64
B = 16384
DEG = 32
TOT = B * DEG + B
NW = 32
PER_W = TOT // NW
CH = 1536
NCH = PER_W // CH

BB2 = 128


TOT2 = TOT // 2
PER_W2 = PER_W // 2
CH2 = CH // 2


def _sc_gather(idx_even, idx_odd, table):
    mesh = plsc.VectorSubcoreMesh(core_axis_name="c", subcore_axis_name="s")

    @functools.partial(
        pl.kernel,
        mesh=mesh,
        out_type=jax.ShapeDtypeStruct((TOT2, 2 * D), jnp.float32),
        scratch_types=[
            pltpu.VMEM((CH2,), jnp.int32),
            pltpu.VMEM((CH2,), jnp.int32),
            pltpu.VMEM((CH2, D), jnp.float32),
            pltpu.VMEM((CH2, D), jnp.float32),
            pltpu.SemaphoreType.DMA,
        ],
        compiler_params=pltpu.CompilerParams(use_tc_tiling_on_sc=False),
    )
    def gather_k(ie_hbm, io_hbm, tab_hbm, out_hbm, ie_v, io_v, re_v, ro_v,
                 sem):
        wid = lax.axis_index("s") * 2 + lax.axis_index("c")
        base = wid * PER_W2

        def body(i, carry):
            off = base + i * CH2
            pltpu.sync_copy(ie_hbm.at[pl.ds(off, CH2)], ie_v)
            pltpu.sync_copy(io_hbm.at[pl.ds(off, CH2)], io_v)
            c1 = pltpu.async_copy(tab_hbm.at[ie_v], re_v, sem)
            c2 = pltpu.async_copy(tab_hbm.at[io_v], ro_v, sem)
            c1.wait()
            c2.wait()
            pltpu.sync_copy(re_v, out_hbm.at[pl.ds(off, CH2), pl.ds(0, D)])
            pltpu.sync_copy(ro_v, out_hbm.at[pl.ds(off, CH2), pl.ds(D, D)])
            return carry

        lax.fori_loop(0, NCH, body, 0)

    return gather_k(idx_even, idx_odd, table)


def _mlp_body(e_ref, u_ref, w1a_ref, w1b_ref, b1_ref, w2_ref, b2_ref,
              w3e_ref, w3o_ref, o_ref):
    uw = jnp.dot(u_ref[...], w1b_ref[...],
                 preferred_element_type=jnp.float32) + b1_ref[...]
    E = e_ref[...]
    X = E.reshape(DEG * BB2, 2 * D)
    UW = jnp.broadcast_to(uw[None], (DEG, BB2, 2 * D)).reshape(DEG * BB2, 2 * D)
    H = jnp.maximum(jnp.dot(X, w1a_ref[...],
                            preferred_element_type=jnp.float32) + UW, 0.0)
    H = jnp.maximum(jnp.dot(H, w2_ref[...],
                            preferred_element_type=jnp.float32) + b2_ref[...], 0.0)
    Se = jnp.dot(H, w3e_ref[...],
                 preferred_element_type=jnp.float32).reshape(DEG, BB2, 1)
    So = jnp.dot(H, w3o_ref[...],
                 preferred_element_type=jnp.float32).reshape(DEG, BB2, 1)
    out_halves = []
    for S3, lo in ((Se, 0), (So, D)):
        m = S3[0]
        for n in range(1, DEG):
            m = jnp.maximum(m, S3[n])
        es = [jnp.exp(S3[n] - m) for n in range(DEG)]
        den = es[0]
        for n in range(1, DEG):
            den = den + es[n]
        inv = 1.0 / den
        acc = (es[0] * inv) * E[0, :, lo:lo + D]
        for n in range(1, DEG):
            acc = acc + (es[n] * inv) * E[n, :, lo:lo + D]
        out_halves.append(acc)
    o_ref[...] = jnp.concatenate(out_halves, axis=1)


def _tc_mlp(e2, u2, w1a_bd, w1b_bd, b1_2, w2_bd, b2_2, w3e, w3o):
    grid = (B // 2 // BB2,)
    full = lambda shape: pl.BlockSpec(shape, lambda i: (0,) * len(shape))
    return pl.pallas_call(
        _mlp_body,
        grid=grid,
        in_specs=[
            pl.BlockSpec((DEG, BB2, 2 * D), lambda i: (0, i, 0)),
            pl.BlockSpec((BB2, 2 * D), lambda i: (i, 0)),
            full((2 * D, 2 * D)), full((2 * D, 2 * D)), full((1, 2 * D)),
            full((2 * D, 2 * D)), full((1, 2 * D)),
            full((2 * D, 1)), full((2 * D, 1)),
        ],
        out_specs=pl.BlockSpec((BB2, 2 * D), lambda i: (i, 0)),
        out_shape=jax.ShapeDtypeStruct((B // 2, 2 * D), jnp.float32),
    )(e2, u2, w1a_bd, w1b_bd, b1_2, w2_bd, b2_2, w3e, w3o)


def kernel(nodes, to_neighs, table, W1, b1, W2, b2, W3, b3):
    idx_all = jnp.concatenate(
        [to_neighs.T.reshape(-1), nodes]).astype(jnp.int32)
    gathered = _sc_gather(idx_all[0::2], idx_all[1::2], table)
    e2 = gathered[: B * DEG // 2].reshape(DEG, B // 2, 2 * D)
    u2 = gathered[B * DEG // 2:]
    zz = jnp.zeros((D, D), jnp.float32)
    bd = lambda w: jnp.block([[w, zz], [zz, w]])
    w1a_bd = bd(W1[:, :D].T)
    w1b_bd = bd(W1[:, D:].T)
    w2_bd = bd(W2.T)
    zcol = jnp.zeros((D, 1), jnp.float32)
    w3e = jnp.concatenate([W3.T, zcol], axis=0)
    w3o = jnp.concatenate([zcol, W3.T], axis=0)
    b1_2 = jnp.tile(b1, 2).reshape(1, 2 * D)
    b2_2 = jnp.tile(b2, 2).reshape(1, 2 * D)
    out2 = _tc_mlp(e2, u2, w1a_bd, w1b_bd, b1_2, w2_bd, b2_2, w3e, w3o)
    return out2.reshape(B, D)
